# baseline (speedup 1.0000x reference)
import functools

import jax
import jax.numpy as jnp
from jax import lax
from jax.experimental import pallas as pl
from jax.experimental.pallas import tpu as pltpu
from jax.experimental.pallas import tpu_sc as plsc

_IDX_MINOR = 128


def _make_lookup(B, S, V, D):
    info = plsc.get_sparse_core_info()
    ncores = info.num_cores
    nsub = info.num_subcores
    lanes = info.num_lanes
    nw = ncores * nsub
    seqs_per_w = B // nw
    spr = S // _IDX_MINOR
    n_idx = seqs_per_w * spr

    mesh = plsc.VectorSubcoreMesh(core_axis_name="c", subcore_axis_name="s")

    @functools.partial(
        pl.kernel,
        out_type=jax.ShapeDtypeStruct((B, D, spr, _IDX_MINOR), jnp.float32),
        mesh=mesh,
        compiler_params=pltpu.CompilerParams(use_tc_tiling_on_sc=False),
        scratch_types=[
            pltpu.VMEM((n_idx, _IDX_MINOR), jnp.int32),
            pltpu.VMEM((n_idx, _IDX_MINOR), jnp.float32),
            pltpu.VMEM((D, spr, _IDX_MINOR), jnp.float32),
            pltpu.VMEM_SHARED((V,), jnp.float32),
            pltpu.SemaphoreType.DMA,
            pltpu.SemaphoreType.DMA,
            pltpu.SemaphoreType.DMA,
        ],
    )
    def lookup(
        x_hbm, xf32_hbm, tokT_hbm, posT_hbm, out_hbm,
        idx_v, obuf_v, pos_v, spm, ssem, gsem, wsem,
    ):
        cid = lax.axis_index("c")
        sid = lax.axis_index("s")
        wid = cid * nsub + sid
        base_seq = wid * seqs_per_w

        pltpu.sync_copy(x_hbm.at[pl.ds(wid * n_idx, n_idx)], idx_v)
        pltpu.sync_copy(posT_hbm, pos_v)

        @pl.when(sid == 0)
        def _():
            pltpu.async_copy(tokT_hbm.at[0], spm, ssem)

        def step(cc, carry):
            @pl.when(sid == 0)
            def _():
                pltpu.make_async_copy(tokT_hbm.at[0], spm, ssem).wait()

            plsc.subcore_barrier()

            @pl.when(cc >= 1)
            def _():
                pltpu.make_async_copy(
                    xf32_hbm.at[pl.ds(0, n_idx)], obuf_v, wsem
                ).wait()

            def fire_g(j, c):
                pltpu.async_copy(spm.at[idx_v.at[j]], obuf_v.at[j], gsem)
                return c

            lax.fori_loop(0, n_idx, fire_g, 0)
            pltpu.make_async_copy(
                xf32_hbm.at[pl.ds(0, n_idx)], obuf_v, gsem
            ).wait()

            plsc.subcore_barrier()

            @pl.when((sid == 0) & (cc + 1 < D))
            def _():
                pltpu.async_copy(tokT_hbm.at[cc + 1], spm, ssem)

            def add_body(j, c):
                for k in range(spr):
                    for h in range(_IDX_MINOR // lanes):
                        sl = pl.ds(h * lanes, lanes)
                        plsc.addupdate(
                            obuf_v.at[j * spr + k, sl], pos_v[cc, k, sl]
                        )
                return c

            lax.fori_loop(0, seqs_per_w, add_body, 0)

            def fire_w(j, c):
                pltpu.async_copy(
                    obuf_v.at[pl.ds(j * spr, spr)],
                    out_hbm.at[base_seq + j, cc],
                    wsem,
                )
                return c

            lax.fori_loop(0, seqs_per_w, fire_w, 0)
            return carry

        lax.fori_loop(0, D, step, 0)

        pltpu.make_async_copy(xf32_hbm.at[pl.ds(0, n_idx)], obuf_v, wsem).wait()

    return lookup


def kernel(x, token_table, pos_table):
    B, S = x.shape
    V, D = token_table.shape
    xf = x.reshape(B * S // _IDX_MINOR, _IDX_MINOR).astype(jnp.int32)
    lookup = _make_lookup(B, S, V, D)
    out = lookup(
        xf,
        jax.lax.bitcast_convert_type(xf, jnp.float32),
        token_table.T,
        pos_table.T.reshape(D, S // _IDX_MINOR, _IDX_MINOR),
    )
    return out.reshape(B, D, S).transpose(0, 2, 1)


# speedup vs baseline: 1.0031x; 1.0031x over previous
import functools

import jax
import jax.numpy as jnp
from jax import lax
from jax.experimental import pallas as pl
from jax.experimental.pallas import tpu as pltpu
from jax.experimental.pallas import tpu_sc as plsc

_IDX_MINOR = 128


def _make_lookup(B, S, V, D):
    info = plsc.get_sparse_core_info()
    ncores = info.num_cores
    nsub = info.num_subcores
    lanes = info.num_lanes
    nw = ncores * nsub
    seqs_per_w = B // nw
    spr = S // _IDX_MINOR
    n_idx = seqs_per_w * spr

    mesh = plsc.VectorSubcoreMesh(core_axis_name="c", subcore_axis_name="s")

    @functools.partial(
        pl.kernel,
        out_type=jax.ShapeDtypeStruct((B, D, S), jnp.float32),
        mesh=mesh,
        compiler_params=pltpu.CompilerParams(use_tc_tiling_on_sc=False),
        scratch_types=[
            pltpu.VMEM((n_idx * _IDX_MINOR,), jnp.int32),
            pltpu.VMEM((n_idx * _IDX_MINOR,), jnp.float32),
            pltpu.VMEM((D, S), jnp.float32),
            pltpu.VMEM_SHARED((V,), jnp.float32),
            pltpu.SemaphoreType.DMA,
            pltpu.SemaphoreType.DMA,
            pltpu.SemaphoreType.DMA,
        ],
    )
    def lookup(
        x_hbm, xf32_hbm, tokT_hbm, posT_hbm, out_hbm,
        idx_v, obuf_v, pos_v, spm, ssem, gsem, wsem,
    ):
        cid = lax.axis_index("c")
        sid = lax.axis_index("s")
        wid = cid * nsub + sid
        base_seq = wid * seqs_per_w

        nw_idx = n_idx * _IDX_MINOR
        pltpu.sync_copy(x_hbm.at[pl.ds(wid * nw_idx, nw_idx)], idx_v)
        pltpu.sync_copy(posT_hbm, pos_v)

        @pl.when(sid == 0)
        def _():
            pltpu.async_copy(tokT_hbm.at[0], spm, ssem)

        def step(cc, carry):
            @pl.when(sid == 0)
            def _():
                pltpu.make_async_copy(tokT_hbm.at[0], spm, ssem).wait()

            plsc.subcore_barrier()

            @pl.when(cc >= 1)
            def _():
                pltpu.make_async_copy(
                    xf32_hbm.at[pl.ds(0, nw_idx)], obuf_v, wsem
                ).wait()

            pltpu.async_copy(spm.at[idx_v], obuf_v, gsem)
            pltpu.make_async_copy(
                xf32_hbm.at[pl.ds(0, nw_idx)], obuf_v, gsem
            ).wait()

            plsc.subcore_barrier()

            @pl.when((sid == 0) & (cc + 1 < D))
            def _():
                pltpu.async_copy(tokT_hbm.at[cc + 1], spm, ssem)

            def add_body(j, c):
                for k in range(S // lanes):
                    sl = pl.ds(k * lanes, lanes)
                    plsc.addupdate(
                        obuf_v.at[pl.ds(j * S + k * lanes, lanes)],
                        pos_v[cc, sl],
                    )
                return c

            lax.fori_loop(0, seqs_per_w, add_body, 0)

            def fire_w(j, c):
                pltpu.async_copy(
                    obuf_v.at[pl.ds(j * S, S)],
                    out_hbm.at[base_seq + j, cc],
                    wsem,
                )
                return c

            lax.fori_loop(0, seqs_per_w, fire_w, 0)
            return carry

        lax.fori_loop(0, D, step, 0)

        pltpu.make_async_copy(xf32_hbm.at[pl.ds(0, nw_idx)], obuf_v, wsem).wait()

    return lookup


def kernel(x, token_table, pos_table):
    B, S = x.shape
    V, D = token_table.shape
    xf = x.reshape(B * S).astype(jnp.int32)
    lookup = _make_lookup(B, S, V, D)
    out = lookup(
        xf,
        jax.lax.bitcast_convert_type(xf, jnp.float32),
        token_table.T,
        pos_table.T,
    )
    return out.transpose(0, 2, 1)
